# Initial kernel scaffold; baseline (speedup 1.0000x reference)
import jax
import jax.numpy as jnp
from jax.experimental import pallas as pl


def kernel(features, h3_nodes, enc_edge_attr, lat_edge_attr, params, enc_edge_index, lat_edge_index):
    raise NotImplementedError("write your pallas kernel here")


# speedup vs baseline: 3.0490x; 3.0490x over previous
import functools

import jax
import jax.numpy as jnp
from jax import lax
from jax.experimental import pallas as pl
from jax.experimental.pallas import tpu as pltpu
from jax.experimental.pallas import tpu_sc as plsc

N_LATLON = 65160
N_GRAPH = 5882
IN_DIM = 78
D = 128

E_PAD = 65280
G_PAD = 5888
LAT_E = N_GRAPH * 7
LAT_PAD = 41472

_NC, _NS = 2, 16
_NW = _NC * _NS
PER_W = E_PAD // _NW
CH = 120
NCH = PER_W // CH
ROWS_PER_TILE = G_PAD // _NS

_f32 = jnp.float32


def _ln(x, g, b):
    mu = jnp.mean(x, axis=-1, keepdims=True)
    xc = x - mu
    var = jnp.mean(xc * xc, axis=-1, keepdims=True)
    return xc * lax.rsqrt(var + 1e-5) * g + b


def _dot(a, b):
    return jnp.dot(a, b, preferred_element_type=_f32)


def _h3_encoder_body(x_ref, w1, w2, w3, b1, b2, b3, g, b, out_ref):
    h = jax.nn.silu(_dot(x_ref[...], w1[...]) + b1[...])
    h = jax.nn.silu(_dot(h, w2[...]) + b2[...])
    h = _dot(h, w3[...]) + b3[...]
    out_ref[...] = _ln(h, g[...], b[...])


def _edge_pipeline_body(feat_ref, attr_ref, dst_ref,
                        nw1, nw2, nw3, nb1, nb2, nb3, ng, nb,
                        ew1, ew2, ew3, eb1, eb2, eb3, eg, eb,
                        gw1s, gw1d, gw1e, gw2, gw3, gb1, gb2, gb3, gg, gb,
                        out_ref):
    h = jax.nn.silu(_dot(feat_ref[...], nw1[...]) + nb1[...])
    h = jax.nn.silu(_dot(h, nw2[...]) + nb2[...])
    h = _dot(h, nw3[...]) + nb3[...]
    src = _ln(h, ng[...], nb[...])
    a = attr_ref[...]
    eh = jax.nn.silu(a[:, 0:1] * ew1[0:1, :] + a[:, 1:2] * ew1[1:2, :] + eb1[...])
    eh = jax.nn.silu(_dot(eh, ew2[...]) + eb2[...])
    eh = _dot(eh, ew3[...]) + eb3[...]
    e = _ln(eh, eg[...], eb[...])
    d = dst_ref[...]
    u = jax.nn.silu(_dot(src, gw1s[...]) + _dot(d, gw1d[...]) + _dot(e, gw1e[...]) + gb1[...])
    u = jax.nn.silu(_dot(u, gw2[...]) + gb2[...])
    u = _dot(u, gw3[...]) + gb3[...]
    out_ref[...] = _ln(u, gg[...], gb[...]) + e


def _node_update_body(h3_ref, agg_ref,
                      w1n, w1a, w2, w3, b1, b2, b3, g, b, out_ref):
    h3 = h3_ref[...]
    agg = agg_ref[0] + agg_ref[1]
    h = jax.nn.silu(_dot(h3, w1n[...]) + _dot(agg, w1a[...]) + b1[...])
    h = jax.nn.silu(_dot(h, w2[...]) + b2[...])
    h = _dot(h, w3[...]) + b3[...]
    out_ref[...] = _ln(h, g[...], b[...]) + h3


def _lat_encoder_body(attr_ref, w1, w2, w3, b1, b2, b3, g, b, out_ref):
    a = attr_ref[...]
    h = jax.nn.silu(a[:, 0:1] * w1[0:1, :] + a[:, 1:2] * w1[1:2, :] + b1[...])
    h = jax.nn.silu(_dot(h, w2[...]) + b2[...])
    h = _dot(h, w3[...]) + b3[...]
    out_ref[...] = _ln(h, g[...], b[...])


def _const_spec(shape):
    return pl.BlockSpec(shape, lambda i: tuple(0 for _ in shape))


def _row_spec(blk, cols):
    return pl.BlockSpec((blk, cols), lambda i: (i, 0))


def _make_gather():
    mesh = plsc.VectorSubcoreMesh(core_axis_name="c", subcore_axis_name="s",
                                  num_cores=_NC, num_subcores=_NS)

    @functools.partial(
        pl.kernel,
        out_type=jax.ShapeDtypeStruct((E_PAD, D), _f32),
        mesh=mesh,
        scratch_types=[
            pltpu.VMEM((CH,), jnp.int32),
            pltpu.VMEM((CH, D), _f32),
            pltpu.SemaphoreType.DMA,
        ],
    )
    def gather_k(table, idx, out, idx_v, rows_v, sem):
        wid = lax.axis_index("s") * _NC + lax.axis_index("c")
        base = wid * PER_W
        for c in range(NCH):
            off = base + c * CH
            pltpu.sync_copy(idx.at[pl.ds(off, CH)], idx_v)
            pltpu.async_copy(table.at[idx_v], rows_v, sem).wait()
            pltpu.sync_copy(rows_v, out.at[pl.ds(off, CH)])

    return gather_k


def _make_scatter():
    mesh = plsc.VectorSubcoreMesh(core_axis_name="c", subcore_axis_name="s",
                                  num_cores=_NC, num_subcores=_NS)

    @functools.partial(
        pl.kernel,
        out_type=jax.ShapeDtypeStruct((_NC, G_PAD, D), _f32),
        mesh=mesh,
        scratch_types=[
            pltpu.VMEM((CH,), jnp.int32),
            pltpu.VMEM((CH, D), _f32),
            pltpu.VMEM_SHARED((G_PAD, D), _f32),
        ],
    )
    def scatter_k(eupd, idx, zeros, out, idx_v, rows_v, agg_sh):
        cid = lax.axis_index("c")
        sid = lax.axis_index("s")
        wid = sid * _NC + cid
        r0 = sid * ROWS_PER_TILE
        pltpu.sync_copy(zeros.at[pl.ds(r0, ROWS_PER_TILE)],
                        agg_sh.at[pl.ds(r0, ROWS_PER_TILE)])
        plsc.subcore_barrier()
        base = wid * PER_W
        for c in range(NCH):
            off = base + c * CH
            pltpu.sync_copy(idx.at[pl.ds(off, CH)], idx_v)
            pltpu.sync_copy(eupd.at[pl.ds(off, CH)], rows_v)
            pltpu.sync_copy(rows_v, agg_sh.at[idx_v], add=True)
        plsc.subcore_barrier()
        pltpu.sync_copy(agg_sh.at[pl.ds(r0, ROWS_PER_TILE)],
                        out.at[cid, pl.ds(r0, ROWS_PER_TILE)])

    return scatter_k


_gather = functools.cache(_make_gather)
_scatter = functools.cache(_make_scatter)


def _mlp_args(p):
    ws = [w for w in p["Ws"]]
    bs = [b.reshape(1, D) for b in p["bs"]]
    return ws, bs, p["g"].reshape(1, D), p["b"].reshape(1, D)


def kernel(features, h3_nodes, enc_edge_attr, lat_edge_attr, params,
           enc_edge_index, lat_edge_index):
    feats = features.reshape(N_LATLON, IN_DIM)
    feats_pad = jnp.pad(feats, ((0, E_PAD - N_LATLON), (0, 0)))
    attr_pad = jnp.pad(enc_edge_attr, ((0, E_PAD - N_LATLON), (0, 0)))
    h3_pad = jnp.pad(h3_nodes, ((0, G_PAD - N_GRAPH), (0, 0)))
    lat_pad = jnp.pad(lat_edge_attr, ((0, LAT_PAD - LAT_E), (0, 0)))
    idx = enc_edge_index[1] - N_LATLON
    idx_pad = jnp.pad(idx, (0, E_PAD - N_LATLON), constant_values=N_GRAPH)

    nws, nbs, ng, nb = _mlp_args(params["node_encoder"])
    ews, ebs, eg, eb = _mlp_args(params["edge_encoder"])
    lws, lbs, lg, lb = _mlp_args(params["latent_edge_encoder"])
    gws, gbs, gg, gb = _mlp_args(params["gp_edge_mlp"])
    gw1s, gw1d, gw1e = gws[0][:D], gws[0][D:2 * D], gws[0][2 * D:]
    pws, pbs, pg, pb = _mlp_args(params["gp_node_mlp"])
    pw1n, pw1a = pws[0][:D], pws[0][D:]

    h3_emb = pl.pallas_call(
        _h3_encoder_body,
        grid=(1,),
        in_specs=[_const_spec((G_PAD, IN_DIM)),
                  _const_spec((IN_DIM, D)), _const_spec((D, D)), _const_spec((D, D)),
                  _const_spec((1, D)), _const_spec((1, D)), _const_spec((1, D)),
                  _const_spec((1, D)), _const_spec((1, D))],
        out_specs=_const_spec((G_PAD, D)),
        out_shape=jax.ShapeDtypeStruct((G_PAD, D), _f32),
    )(h3_pad, nws[0], nws[1], nws[2], nbs[0], nbs[1], nbs[2], ng, nb)

    dst_emb = _gather()(h3_emb, idx_pad)

    BLK = 640
    e_upd = pl.pallas_call(
        _edge_pipeline_body,
        grid=(E_PAD // BLK,),
        in_specs=[_row_spec(BLK, IN_DIM), _row_spec(BLK, 2), _row_spec(BLK, D),
                  _const_spec((IN_DIM, D)), _const_spec((D, D)), _const_spec((D, D)),
                  _const_spec((1, D)), _const_spec((1, D)), _const_spec((1, D)),
                  _const_spec((1, D)), _const_spec((1, D)),
                  _const_spec((2, D)), _const_spec((D, D)), _const_spec((D, D)),
                  _const_spec((1, D)), _const_spec((1, D)), _const_spec((1, D)),
                  _const_spec((1, D)), _const_spec((1, D)),
                  _const_spec((D, D)), _const_spec((D, D)), _const_spec((D, D)),
                  _const_spec((D, D)), _const_spec((D, D)),
                  _const_spec((1, D)), _const_spec((1, D)), _const_spec((1, D)),
                  _const_spec((1, D)), _const_spec((1, D))],
        out_specs=_row_spec(BLK, D),
        out_shape=jax.ShapeDtypeStruct((E_PAD, D), _f32),
    )(feats_pad, attr_pad, dst_emb,
      nws[0], nws[1], nws[2], nbs[0], nbs[1], nbs[2], ng, nb,
      ews[0], ews[1], ews[2], ebs[0], ebs[1], ebs[2], eg, eb,
      gw1s, gw1d, gw1e, gws[1], gws[2], gbs[0], gbs[1], gbs[2], gg, gb)

    zeros = jnp.zeros((G_PAD, D), _f32)
    agg2 = _scatter()(e_upd, idx_pad, zeros)

    out_h3 = pl.pallas_call(
        _node_update_body,
        grid=(1,),
        in_specs=[_const_spec((G_PAD, D)), _const_spec((_NC, G_PAD, D)),
                  _const_spec((D, D)), _const_spec((D, D)), _const_spec((D, D)),
                  _const_spec((D, D)),
                  _const_spec((1, D)), _const_spec((1, D)), _const_spec((1, D)),
                  _const_spec((1, D)), _const_spec((1, D))],
        out_specs=_const_spec((G_PAD, D)),
        out_shape=jax.ShapeDtypeStruct((G_PAD, D), _f32),
    )(h3_emb, agg2, pw1n, pw1a, pws[1], pws[2], pbs[0], pbs[1], pbs[2], pg, pb)

    LBLK = 512
    lat_e = pl.pallas_call(
        _lat_encoder_body,
        grid=(LAT_PAD // LBLK,),
        in_specs=[_row_spec(LBLK, 2),
                  _const_spec((2, D)), _const_spec((D, D)), _const_spec((D, D)),
                  _const_spec((1, D)), _const_spec((1, D)), _const_spec((1, D)),
                  _const_spec((1, D)), _const_spec((1, D))],
        out_specs=_row_spec(LBLK, D),
        out_shape=jax.ShapeDtypeStruct((LAT_PAD, D), _f32),
    )(lat_pad, lws[0], lws[1], lws[2], lbs[0], lbs[1], lbs[2], lg, lb)

    return out_h3[:N_GRAPH], lat_edge_index, lat_e[:LAT_E]


# speedup vs baseline: 3.1702x; 1.0398x over previous
import functools

import jax
import jax.numpy as jnp
from jax import lax
from jax.experimental import pallas as pl
from jax.experimental.pallas import tpu as pltpu
from jax.experimental.pallas import tpu_sc as plsc

N_LATLON = 65160
N_GRAPH = 5882
IN_DIM = 78
D = 128

E_PAD = 65280
G_PAD = 5888
LAT_E = N_GRAPH * 7
LAT_PAD = 41472

_NC, _NS = 2, 16
_NW = _NC * _NS
PER_W = E_PAD // _NW
CH = 120
NCH = PER_W // CH
ROWS_PER_TILE = G_PAD // _NS

_f32 = jnp.float32


def _ln(x, g, b):
    mu = jnp.mean(x, axis=-1, keepdims=True)
    xc = x - mu
    var = jnp.mean(xc * xc, axis=-1, keepdims=True)
    return xc * lax.rsqrt(var + 1e-5) * g + b


def _dot(a, b):
    return jnp.dot(a, b, preferred_element_type=_f32)


def _h3_encoder_body(x_ref, w1, w2, w3, b1, b2, b3, g, b, out_ref):
    h = jax.nn.silu(_dot(x_ref[...], w1[...]) + b1[...])
    h = jax.nn.silu(_dot(h, w2[...]) + b2[...])
    h = _dot(h, w3[...]) + b3[...]
    out_ref[...] = _ln(h, g[...], b[...])


def _edge_pipeline_body(feat_ref, attr_ref, dst_ref,
                        nw1, nw2, nw3, nb1, nb2, nb3, ng, nb,
                        ew1, ew2, ew3, eb1, eb2, eb3, eg, eb,
                        gw1s, gw1d, gw1e, gw2, gw3, gb1, gb2, gb3, gg, gb,
                        out_ref):
    h = jax.nn.silu(_dot(feat_ref[...], nw1[...]) + nb1[...])
    h = jax.nn.silu(_dot(h, nw2[...]) + nb2[...])
    h = _dot(h, nw3[...]) + nb3[...]
    src = _ln(h, ng[...], nb[...])
    a = attr_ref[...]
    eh = jax.nn.silu(a[:, 0:1] * ew1[0:1, :] + a[:, 1:2] * ew1[1:2, :] + eb1[...])
    eh = jax.nn.silu(_dot(eh, ew2[...]) + eb2[...])
    eh = _dot(eh, ew3[...]) + eb3[...]
    e = _ln(eh, eg[...], eb[...])
    d = dst_ref[...]
    u = jax.nn.silu(_dot(src, gw1s[...]) + _dot(d, gw1d[...]) + _dot(e, gw1e[...]) + gb1[...])
    u = jax.nn.silu(_dot(u, gw2[...]) + gb2[...])
    u = _dot(u, gw3[...]) + gb3[...]
    out_ref[...] = _ln(u, gg[...], gb[...]) + e


def _node_update_body(h3_ref, agg_ref,
                      w1n, w1a, w2, w3, b1, b2, b3, g, b, out_ref):
    h3 = h3_ref[...]
    agg = agg_ref[0] + agg_ref[1]
    h = jax.nn.silu(_dot(h3, w1n[...]) + _dot(agg, w1a[...]) + b1[...])
    h = jax.nn.silu(_dot(h, w2[...]) + b2[...])
    h = _dot(h, w3[...]) + b3[...]
    out_ref[...] = _ln(h, g[...], b[...]) + h3


def _lat_encoder_body(attr_ref, w1, w2, w3, b1, b2, b3, g, b, out_ref):
    a = attr_ref[...]
    h = jax.nn.silu(a[:, 0:1] * w1[0:1, :] + a[:, 1:2] * w1[1:2, :] + b1[...])
    h = jax.nn.silu(_dot(h, w2[...]) + b2[...])
    h = _dot(h, w3[...]) + b3[...]
    out_ref[...] = _ln(h, g[...], b[...])


def _const_spec(shape):
    return pl.BlockSpec(shape, lambda i: tuple(0 for _ in shape))


def _row_spec(blk, cols):
    return pl.BlockSpec((blk, cols), lambda i: (i, 0))


def _make_gather():
    mesh = plsc.VectorSubcoreMesh(core_axis_name="c", subcore_axis_name="s",
                                  num_cores=_NC, num_subcores=_NS)

    @functools.partial(
        pl.kernel,
        out_type=jax.ShapeDtypeStruct((E_PAD, D), _f32),
        mesh=mesh,
        scratch_types=[
            pltpu.VMEM((CH,), jnp.int32),
            pltpu.VMEM((CH,), jnp.int32),
            pltpu.VMEM((CH, D), _f32),
            pltpu.VMEM((CH, D), _f32),
            pltpu.SemaphoreType.DMA,
            pltpu.SemaphoreType.DMA,
            pltpu.SemaphoreType.DMA,
            pltpu.SemaphoreType.DMA,
            pltpu.SemaphoreType.DMA,
            pltpu.SemaphoreType.DMA,
        ],
    )
    def gather_k(table, idx, out, idxb0, idxb1, rows0, rows1,
                 i0, i1, g0, g1, w0, w1):
        wid = lax.axis_index("s") * _NC + lax.axis_index("c")
        base = wid * PER_W
        idxb = (idxb0, idxb1)
        rows = (rows0, rows1)
        isem = (i0, i1)
        gsem = (g0, g1)
        wsem = (w0, w1)
        i = [None, None]
        w = [None, None]
        i[0] = pltpu.async_copy(idx.at[pl.ds(base, CH)], idxb[0], isem[0])
        for c in range(NCH):
            s = c % 2
            o = (c + 1) % 2
            if c + 1 < NCH:
                i[o] = pltpu.async_copy(
                    idx.at[pl.ds(base + (c + 1) * CH, CH)], idxb[o], isem[o])
            i[s].wait()
            if w[s] is not None:
                w[s].wait()
            pltpu.async_copy(table.at[idxb[s]], rows[s], gsem[s]).wait()
            w[s] = pltpu.async_copy(rows[s], out.at[pl.ds(base + c * CH, CH)],
                                    wsem[s])
        w[(NCH - 1) % 2].wait()
        if NCH > 1:
            w[NCH % 2].wait()

    return gather_k


def _make_scatter():
    mesh = plsc.VectorSubcoreMesh(core_axis_name="c", subcore_axis_name="s",
                                  num_cores=_NC, num_subcores=_NS)

    @functools.partial(
        pl.kernel,
        out_type=jax.ShapeDtypeStruct((_NC, G_PAD, D), _f32),
        mesh=mesh,
        scratch_types=[
            pltpu.VMEM((CH,), jnp.int32),
            pltpu.VMEM((CH,), jnp.int32),
            pltpu.VMEM((CH, D), _f32),
            pltpu.VMEM((CH, D), _f32),
            pltpu.VMEM_SHARED((G_PAD, D), _f32),
            pltpu.SemaphoreType.DMA,
            pltpu.SemaphoreType.DMA,
            pltpu.SemaphoreType.DMA,
            pltpu.SemaphoreType.DMA,
            pltpu.SemaphoreType.DMA,
            pltpu.SemaphoreType.DMA,
        ],
    )
    def scatter_k(eupd, idx, zeros, out, idxb0, idxb1, rows0, rows1, agg_sh,
                  i0, i1, r0s, r1s, a0, a1):
        cid = lax.axis_index("c")
        sid = lax.axis_index("s")
        wid = sid * _NC + cid
        zr = sid * ROWS_PER_TILE
        pltpu.sync_copy(zeros, agg_sh.at[pl.ds(zr, ROWS_PER_TILE)])
        plsc.subcore_barrier()
        base = wid * PER_W
        idxb = (idxb0, idxb1)
        rows = (rows0, rows1)
        isem = (i0, i1)
        rsem = (r0s, r1s)
        asem = (a0, a1)
        i = [None, None]
        r = [None, None]
        a = [None, None]
        i[0] = pltpu.async_copy(idx.at[pl.ds(base, CH)], idxb[0], isem[0])
        r[0] = pltpu.async_copy(eupd.at[pl.ds(base, CH)], rows[0], rsem[0])
        for c in range(NCH):
            s = c % 2
            o = (c + 1) % 2
            if c + 1 < NCH:
                if a[o] is not None:
                    a[o].wait()
                i[o] = pltpu.async_copy(
                    idx.at[pl.ds(base + (c + 1) * CH, CH)], idxb[o], isem[o])
                r[o] = pltpu.async_copy(
                    eupd.at[pl.ds(base + (c + 1) * CH, CH)], rows[o], rsem[o])
            i[s].wait()
            r[s].wait()
            a[s] = pltpu.async_copy(rows[s], agg_sh.at[idxb[s]], asem[s],
                                    add=True)
        a[(NCH - 1) % 2].wait()
        if NCH > 1:
            a[NCH % 2].wait()
        plsc.subcore_barrier()
        pltpu.sync_copy(agg_sh.at[pl.ds(zr, ROWS_PER_TILE)],
                        out.at[cid, pl.ds(zr, ROWS_PER_TILE)])

    return scatter_k


_gather = functools.cache(_make_gather)
_scatter = functools.cache(_make_scatter)


def _mlp_args(p):
    ws = [w for w in p["Ws"]]
    bs = [b.reshape(1, D) for b in p["bs"]]
    return ws, bs, p["g"].reshape(1, D), p["b"].reshape(1, D)


def kernel(features, h3_nodes, enc_edge_attr, lat_edge_attr, params,
           enc_edge_index, lat_edge_index):
    feats = features.reshape(N_LATLON, IN_DIM)
    h3_pad = jnp.pad(h3_nodes, ((0, G_PAD - N_GRAPH), (0, 0)))
    lat_pad = jnp.pad(lat_edge_attr, ((0, LAT_PAD - LAT_E), (0, 0)))
    idx = enc_edge_index[1] - N_LATLON
    idx_pad = jnp.pad(idx, (0, E_PAD - N_LATLON), constant_values=N_GRAPH)

    nws, nbs, ng, nb = _mlp_args(params["node_encoder"])
    ews, ebs, eg, eb = _mlp_args(params["edge_encoder"])
    lws, lbs, lg, lb = _mlp_args(params["latent_edge_encoder"])
    gws, gbs, gg, gb = _mlp_args(params["gp_edge_mlp"])
    gw1s, gw1d, gw1e = gws[0][:D], gws[0][D:2 * D], gws[0][2 * D:]
    pws, pbs, pg, pb = _mlp_args(params["gp_node_mlp"])
    pw1n, pw1a = pws[0][:D], pws[0][D:]

    h3_emb = pl.pallas_call(
        _h3_encoder_body,
        grid=(1,),
        in_specs=[_const_spec((G_PAD, IN_DIM)),
                  _const_spec((IN_DIM, D)), _const_spec((D, D)), _const_spec((D, D)),
                  _const_spec((1, D)), _const_spec((1, D)), _const_spec((1, D)),
                  _const_spec((1, D)), _const_spec((1, D))],
        out_specs=_const_spec((G_PAD, D)),
        out_shape=jax.ShapeDtypeStruct((G_PAD, D), _f32),
    )(h3_pad, nws[0], nws[1], nws[2], nbs[0], nbs[1], nbs[2], ng, nb)

    dst_emb = _gather()(h3_emb, idx_pad)

    BLK = 640
    e_upd = pl.pallas_call(
        _edge_pipeline_body,
        grid=(E_PAD // BLK,),
        in_specs=[_row_spec(BLK, IN_DIM), _row_spec(BLK, 2), _row_spec(BLK, D),
                  _const_spec((IN_DIM, D)), _const_spec((D, D)), _const_spec((D, D)),
                  _const_spec((1, D)), _const_spec((1, D)), _const_spec((1, D)),
                  _const_spec((1, D)), _const_spec((1, D)),
                  _const_spec((2, D)), _const_spec((D, D)), _const_spec((D, D)),
                  _const_spec((1, D)), _const_spec((1, D)), _const_spec((1, D)),
                  _const_spec((1, D)), _const_spec((1, D)),
                  _const_spec((D, D)), _const_spec((D, D)), _const_spec((D, D)),
                  _const_spec((D, D)), _const_spec((D, D)),
                  _const_spec((1, D)), _const_spec((1, D)), _const_spec((1, D)),
                  _const_spec((1, D)), _const_spec((1, D))],
        out_specs=_row_spec(BLK, D),
        out_shape=jax.ShapeDtypeStruct((E_PAD, D), _f32),
    )(feats, enc_edge_attr, dst_emb,
      nws[0], nws[1], nws[2], nbs[0], nbs[1], nbs[2], ng, nb,
      ews[0], ews[1], ews[2], ebs[0], ebs[1], ebs[2], eg, eb,
      gw1s, gw1d, gw1e, gws[1], gws[2], gbs[0], gbs[1], gbs[2], gg, gb)

    zeros = jnp.zeros((ROWS_PER_TILE, D), _f32)
    agg2 = _scatter()(e_upd, idx_pad, zeros)

    out_h3 = pl.pallas_call(
        _node_update_body,
        grid=(1,),
        in_specs=[_const_spec((G_PAD, D)), _const_spec((_NC, G_PAD, D)),
                  _const_spec((D, D)), _const_spec((D, D)), _const_spec((D, D)),
                  _const_spec((D, D)),
                  _const_spec((1, D)), _const_spec((1, D)), _const_spec((1, D)),
                  _const_spec((1, D)), _const_spec((1, D))],
        out_specs=_const_spec((G_PAD, D)),
        out_shape=jax.ShapeDtypeStruct((G_PAD, D), _f32),
    )(h3_emb, agg2, pw1n, pw1a, pws[1], pws[2], pbs[0], pbs[1], pbs[2], pg, pb)

    LBLK = 512
    lat_e = pl.pallas_call(
        _lat_encoder_body,
        grid=(LAT_PAD // LBLK,),
        in_specs=[_row_spec(LBLK, 2),
                  _const_spec((2, D)), _const_spec((D, D)), _const_spec((D, D)),
                  _const_spec((1, D)), _const_spec((1, D)), _const_spec((1, D)),
                  _const_spec((1, D)), _const_spec((1, D))],
        out_specs=_row_spec(LBLK, D),
        out_shape=jax.ShapeDtypeStruct((LAT_PAD, D), _f32),
    )(lat_pad, lws[0], lws[1], lws[2], lbs[0], lbs[1], lbs[2], lg, lb)

    return out_h3[:N_GRAPH], lat_edge_index, lat_e[:LAT_E]


# speedup vs baseline: 3.4513x; 1.0887x over previous
import functools

import jax
import jax.numpy as jnp
from jax import lax
from jax.experimental import pallas as pl
from jax.experimental.pallas import tpu as pltpu
from jax.experimental.pallas import tpu_sc as plsc

N_LATLON = 65160
N_GRAPH = 5882
IN_DIM = 78
D = 128

E_PAD = 65280
G_PAD = 5888
LAT_E = N_GRAPH * 7
LAT_PAD = 41472

_NC, _NS = 2, 16
_NW = _NC * _NS
PER_W = E_PAD // _NW
CH = 120
NCH = PER_W // CH
ROWS_PER_TILE = G_PAD // _NS

_f32 = jnp.float32


def _ln(x, g, b):
    mu = jnp.mean(x, axis=-1, keepdims=True)
    xc = x - mu
    var = jnp.mean(xc * xc, axis=-1, keepdims=True)
    return xc * lax.rsqrt(var + 1e-5) * g + b


def _dot(a, b):
    return jnp.dot(a, b, preferred_element_type=_f32)


def _h3_encoder_body(x_ref, w1, w2, w3, b1, b2, b3, g, b, out_ref):
    h = jax.nn.silu(_dot(x_ref[...], w1[...]) + b1[...])
    h = jax.nn.silu(_dot(h, w2[...]) + b2[...])
    h = _dot(h, w3[...]) + b3[...]
    out_ref[...] = _ln(h, g[...], b[...])


def _edge_pipeline_body(feat_ref, attr_ref, dst_ref,
                        nw1, nw2, nw3, nb1, nb2, nb3, ng, nb,
                        ew1, ew2, ew3, eb1, eb2, eb3, eg, eb,
                        gw1s, gw1d, gw1e, gw2, gw3, gb1, gb2, gb3, gg, gb,
                        out_ref):
    h = jax.nn.silu(_dot(feat_ref[...], nw1[...]) + nb1[...])
    h = jax.nn.silu(_dot(h, nw2[...]) + nb2[...])
    h = _dot(h, nw3[...]) + nb3[...]
    src = _ln(h, ng[...], nb[...])
    a = attr_ref[...]
    eh = jax.nn.silu(a[:, 0:1] * ew1[0:1, :] + a[:, 1:2] * ew1[1:2, :] + eb1[...])
    eh = jax.nn.silu(_dot(eh, ew2[...]) + eb2[...])
    eh = _dot(eh, ew3[...]) + eb3[...]
    e = _ln(eh, eg[...], eb[...])
    d = dst_ref[...]
    u = jax.nn.silu(_dot(src, gw1s[...]) + _dot(d, gw1d[...]) + _dot(e, gw1e[...]) + gb1[...])
    u = jax.nn.silu(_dot(u, gw2[...]) + gb2[...])
    u = _dot(u, gw3[...]) + gb3[...]
    out_ref[...] = _ln(u, gg[...], gb[...]) + e


def _node_update_body(h3_ref, agg_ref,
                      w1n, w1a, w2, w3, b1, b2, b3, g, b, out_ref):
    h3 = h3_ref[...]
    agg = agg_ref[0] + agg_ref[1]
    h = jax.nn.silu(_dot(h3, w1n[...]) + _dot(agg, w1a[...]) + b1[...])
    h = jax.nn.silu(_dot(h, w2[...]) + b2[...])
    h = _dot(h, w3[...]) + b3[...]
    out_ref[...] = (_ln(h, g[...], b[...]) + h3)[:N_GRAPH]


def _lat_encoder_body(attr_ref, w1, w2, w3, b1, b2, b3, g, b, out_ref):
    a = attr_ref[...]
    h = jax.nn.silu(a[:, 0:1] * w1[0:1, :] + a[:, 1:2] * w1[1:2, :] + b1[...])
    h = jax.nn.silu(_dot(h, w2[...]) + b2[...])
    h = _dot(h, w3[...]) + b3[...]
    out_ref[...] = _ln(h, g[...], b[...])


def _const_spec(shape):
    return pl.BlockSpec(shape, lambda i: tuple(0 for _ in shape))


def _row_spec(blk, cols):
    return pl.BlockSpec((blk, cols), lambda i: (i, 0))


def _make_gather():
    mesh = plsc.VectorSubcoreMesh(core_axis_name="c", subcore_axis_name="s",
                                  num_cores=_NC, num_subcores=_NS)

    @functools.partial(
        pl.kernel,
        out_type=jax.ShapeDtypeStruct((E_PAD, D), _f32),
        mesh=mesh,
        scratch_types=[
            pltpu.VMEM((CH,), jnp.int32),
            pltpu.VMEM((CH,), jnp.int32),
            pltpu.VMEM((CH, D), _f32),
            pltpu.VMEM((CH, D), _f32),
            pltpu.SemaphoreType.DMA,
            pltpu.SemaphoreType.DMA,
            pltpu.SemaphoreType.DMA,
            pltpu.SemaphoreType.DMA,
            pltpu.SemaphoreType.DMA,
            pltpu.SemaphoreType.DMA,
        ],
    )
    def gather_k(table, idx, out, idxb0, idxb1, rows0, rows1,
                 i0, i1, g0, g1, w0, w1):
        wid = lax.axis_index("s") * _NC + lax.axis_index("c")
        base = wid * PER_W
        idxb = (idxb0, idxb1)
        rows = (rows0, rows1)
        isem = (i0, i1)
        gsem = (g0, g1)
        wsem = (w0, w1)
        i = [None, None]
        w = [None, None]
        i[0] = pltpu.async_copy(idx.at[pl.ds(base, CH)], idxb[0], isem[0])
        for c in range(NCH):
            s = c % 2
            o = (c + 1) % 2
            if c + 1 < NCH:
                i[o] = pltpu.async_copy(
                    idx.at[pl.ds(base + (c + 1) * CH, CH)], idxb[o], isem[o])
            i[s].wait()
            if w[s] is not None:
                w[s].wait()
            pltpu.async_copy(table.at[idxb[s]], rows[s], gsem[s]).wait()
            w[s] = pltpu.async_copy(rows[s], out.at[pl.ds(base + c * CH, CH)],
                                    wsem[s])
        w[(NCH - 1) % 2].wait()
        if NCH > 1:
            w[NCH % 2].wait()

    return gather_k


def _make_scatter():
    mesh = plsc.VectorSubcoreMesh(core_axis_name="c", subcore_axis_name="s",
                                  num_cores=_NC, num_subcores=_NS)

    @functools.partial(
        pl.kernel,
        out_type=jax.ShapeDtypeStruct((_NC, G_PAD, D), _f32),
        mesh=mesh,
        scratch_types=[
            pltpu.VMEM((CH,), jnp.int32),
            pltpu.VMEM((CH,), jnp.int32),
            pltpu.VMEM((CH, D), _f32),
            pltpu.VMEM((CH, D), _f32),
            pltpu.VMEM_SHARED((G_PAD, D), _f32),
            pltpu.SemaphoreType.DMA,
            pltpu.SemaphoreType.DMA,
            pltpu.SemaphoreType.DMA,
            pltpu.SemaphoreType.DMA,
            pltpu.SemaphoreType.DMA,
            pltpu.SemaphoreType.DMA,
        ],
    )
    def scatter_k(eupd, idx, zeros, out, idxb0, idxb1, rows0, rows1, agg_sh,
                  i0, i1, r0s, r1s, a0, a1):
        cid = lax.axis_index("c")
        sid = lax.axis_index("s")
        wid = sid * _NC + cid
        zr = sid * ROWS_PER_TILE
        pltpu.sync_copy(zeros, agg_sh.at[pl.ds(zr, ROWS_PER_TILE)])
        plsc.subcore_barrier()
        base = wid * PER_W
        idxb = (idxb0, idxb1)
        rows = (rows0, rows1)
        isem = (i0, i1)
        rsem = (r0s, r1s)
        asem = (a0, a1)
        i = [None, None]
        r = [None, None]
        a = [None, None]
        i[0] = pltpu.async_copy(idx.at[pl.ds(base, CH)], idxb[0], isem[0])
        r[0] = pltpu.async_copy(eupd.at[pl.ds(base, CH)], rows[0], rsem[0])
        for c in range(NCH):
            s = c % 2
            o = (c + 1) % 2
            if c + 1 < NCH:
                if a[o] is not None:
                    a[o].wait()
                i[o] = pltpu.async_copy(
                    idx.at[pl.ds(base + (c + 1) * CH, CH)], idxb[o], isem[o])
                r[o] = pltpu.async_copy(
                    eupd.at[pl.ds(base + (c + 1) * CH, CH)], rows[o], rsem[o])
            i[s].wait()
            r[s].wait()
            a[s] = pltpu.async_copy(rows[s], agg_sh.at[idxb[s]], asem[s],
                                    add=True)
        a[(NCH - 1) % 2].wait()
        if NCH > 1:
            a[NCH % 2].wait()
        plsc.subcore_barrier()
        pltpu.sync_copy(agg_sh.at[pl.ds(zr, ROWS_PER_TILE)],
                        out.at[cid, pl.ds(zr, ROWS_PER_TILE)])

    return scatter_k


_gather = functools.cache(_make_gather)
_scatter = functools.cache(_make_scatter)


def _mlp_args(p):
    ws = [w for w in p["Ws"]]
    bs = [b.reshape(1, D) for b in p["bs"]]
    return ws, bs, p["g"].reshape(1, D), p["b"].reshape(1, D)


def kernel(features, h3_nodes, enc_edge_attr, lat_edge_attr, params,
           enc_edge_index, lat_edge_index):
    feats = features.reshape(N_LATLON, IN_DIM)
    h3_pad = jnp.pad(h3_nodes, ((0, G_PAD - N_GRAPH), (0, 0)))
    idx = enc_edge_index[1] - N_LATLON
    idx_pad = jnp.pad(idx, (0, E_PAD - N_LATLON), constant_values=N_GRAPH)

    nws, nbs, ng, nb = _mlp_args(params["node_encoder"])
    ews, ebs, eg, eb = _mlp_args(params["edge_encoder"])
    lws, lbs, lg, lb = _mlp_args(params["latent_edge_encoder"])
    gws, gbs, gg, gb = _mlp_args(params["gp_edge_mlp"])
    gw1s, gw1d, gw1e = gws[0][:D], gws[0][D:2 * D], gws[0][2 * D:]
    pws, pbs, pg, pb = _mlp_args(params["gp_node_mlp"])
    pw1n, pw1a = pws[0][:D], pws[0][D:]

    h3_emb = pl.pallas_call(
        _h3_encoder_body,
        grid=(1,),
        in_specs=[_const_spec((G_PAD, IN_DIM)),
                  _const_spec((IN_DIM, D)), _const_spec((D, D)), _const_spec((D, D)),
                  _const_spec((1, D)), _const_spec((1, D)), _const_spec((1, D)),
                  _const_spec((1, D)), _const_spec((1, D))],
        out_specs=_const_spec((G_PAD, D)),
        out_shape=jax.ShapeDtypeStruct((G_PAD, D), _f32),
    )(h3_pad, nws[0], nws[1], nws[2], nbs[0], nbs[1], nbs[2], ng, nb)

    dst_emb = _gather()(h3_emb, idx_pad)

    BLK = 640
    e_upd = pl.pallas_call(
        _edge_pipeline_body,
        grid=(E_PAD // BLK,),
        in_specs=[_row_spec(BLK, IN_DIM), _row_spec(BLK, 2), _row_spec(BLK, D),
                  _const_spec((IN_DIM, D)), _const_spec((D, D)), _const_spec((D, D)),
                  _const_spec((1, D)), _const_spec((1, D)), _const_spec((1, D)),
                  _const_spec((1, D)), _const_spec((1, D)),
                  _const_spec((2, D)), _const_spec((D, D)), _const_spec((D, D)),
                  _const_spec((1, D)), _const_spec((1, D)), _const_spec((1, D)),
                  _const_spec((1, D)), _const_spec((1, D)),
                  _const_spec((D, D)), _const_spec((D, D)), _const_spec((D, D)),
                  _const_spec((D, D)), _const_spec((D, D)),
                  _const_spec((1, D)), _const_spec((1, D)), _const_spec((1, D)),
                  _const_spec((1, D)), _const_spec((1, D))],
        out_specs=_row_spec(BLK, D),
        out_shape=jax.ShapeDtypeStruct((E_PAD, D), _f32),
    )(feats, enc_edge_attr, dst_emb,
      nws[0], nws[1], nws[2], nbs[0], nbs[1], nbs[2], ng, nb,
      ews[0], ews[1], ews[2], ebs[0], ebs[1], ebs[2], eg, eb,
      gw1s, gw1d, gw1e, gws[1], gws[2], gbs[0], gbs[1], gbs[2], gg, gb)

    zeros = jnp.zeros((ROWS_PER_TILE, D), _f32)
    agg2 = _scatter()(e_upd, idx_pad, zeros)

    out_h3 = pl.pallas_call(
        _node_update_body,
        grid=(1,),
        in_specs=[_const_spec((G_PAD, D)), _const_spec((_NC, G_PAD, D)),
                  _const_spec((D, D)), _const_spec((D, D)), _const_spec((D, D)),
                  _const_spec((D, D)),
                  _const_spec((1, D)), _const_spec((1, D)), _const_spec((1, D)),
                  _const_spec((1, D)), _const_spec((1, D))],
        out_specs=_const_spec((N_GRAPH, D)),
        out_shape=jax.ShapeDtypeStruct((N_GRAPH, D), _f32),
    )(h3_emb, agg2, pw1n, pw1a, pws[1], pws[2], pbs[0], pbs[1], pbs[2], pg, pb)

    LBLK = 512
    lat_e = pl.pallas_call(
        _lat_encoder_body,
        grid=(pl.cdiv(LAT_E, LBLK),),
        in_specs=[_row_spec(LBLK, 2),
                  _const_spec((2, D)), _const_spec((D, D)), _const_spec((D, D)),
                  _const_spec((1, D)), _const_spec((1, D)), _const_spec((1, D)),
                  _const_spec((1, D)), _const_spec((1, D))],
        out_specs=_row_spec(LBLK, D),
        out_shape=jax.ShapeDtypeStruct((LAT_E, D), _f32),
    )(lat_edge_attr, lws[0], lws[1], lws[2], lbs[0], lbs[1], lbs[2], lg, lb)

    return out_h3, lat_edge_index, lat_e
